# baseline (speedup 1.0000x reference)
import functools
import jax
import jax.numpy as jnp
from jax import lax
from jax.experimental import pallas as pl
from jax.experimental.pallas import tpu as pltpu
from jax.experimental.pallas import tpu_sc as plsc

B, N, F = 16, 4096, 512
E = 64
HI = 2048
HALF = N // 2
TE = E // 2
LANES = 16
CPR = F // LANES

_mesh = plsc.VectorSubcoreMesh(core_axis_name="c", subcore_axis_name="s")


@functools.partial(
    pl.kernel,
    mesh=_mesh,
    out_type=jax.ShapeDtypeStruct((B, N + E, F), jnp.float32),
    scratch_types=[
        pltpu.VMEM((TE, F), jnp.float32),
        pltpu.VMEM((TE, F), jnp.float32),
        pltpu.SemaphoreType.DMA,
        pltpu.SemaphoreType.DMA,
    ],
)
def _sc_unpool(x_hbm, out_hbm, lo_v, hi_v, sem0, sem1):
    cid = lax.axis_index("c")
    sid = lax.axis_index("s")
    b = sid
    h = cid
    r0 = h * HALF
    t0 = h * TE

    copy = pltpu.make_async_copy(
        x_hbm.at[b, pl.ds(r0, HALF), :],
        out_hbm.at[b, pl.ds(r0, HALF), :],
        sem0,
    )
    copy.start()

    ld_lo = pltpu.make_async_copy(x_hbm.at[b, pl.ds(t0, TE), :], lo_v, sem1)
    ld_hi = pltpu.make_async_copy(x_hbm.at[b, pl.ds(HI + t0, TE), :], hi_v, sem1)
    ld_lo.start()
    ld_hi.start()
    ld_lo.wait()
    ld_hi.wait()

    def _row(r, carry):
        def _col(ci, c2):
            cc = ci * LANES
            lo_v[r, pl.ds(cc, LANES)] = 0.5 * (
                lo_v[r, pl.ds(cc, LANES)] + hi_v[r, pl.ds(cc, LANES)]
            )
            return c2
        return lax.fori_loop(0, CPR, _col, carry)

    lax.fori_loop(0, TE, _row, 0)

    st = pltpu.make_async_copy(lo_v, out_hbm.at[b, pl.ds(N + t0, TE), :], sem1)
    st.start()
    st.wait()
    copy.wait()


def kernel(inputs):
    return _sc_unpool(inputs)


# speedup vs baseline: 33.3234x; 33.3234x over previous
import functools
import jax
import jax.numpy as jnp
from jax import lax
from jax.experimental import pallas as pl
from jax.experimental.pallas import tpu as pltpu
from jax.experimental.pallas import tpu_sc as plsc

B, N, F = 16, 4096, 512
E = 64
HI = 2048
HALF = N // 2
TE = E // 2
LANES = 16
CPR = F // LANES
C = 64
NCHUNK = HALF // C

_mesh = plsc.VectorSubcoreMesh(core_axis_name="c", subcore_axis_name="s")


@functools.partial(
    pl.kernel,
    mesh=_mesh,
    out_type=jax.ShapeDtypeStruct((B, N + E, F), jnp.float32),
    scratch_types=[
        pltpu.VMEM((C, F), jnp.float32),
        pltpu.VMEM((C, F), jnp.float32),
        pltpu.VMEM((TE, F), jnp.float32),
        pltpu.VMEM((TE, F), jnp.float32),
        pltpu.SemaphoreType.DMA,
        pltpu.SemaphoreType.DMA,
        pltpu.SemaphoreType.DMA,
        pltpu.SemaphoreType.DMA,
        pltpu.SemaphoreType.DMA,
    ],
)
def _sc_unpool(x_hbm, out_hbm, buf0, buf1, lo_v, hi_v,
               in0_sem, in1_sem, out0_sem, out1_sem, tail_sem):
    cid = lax.axis_index("c")
    sid = lax.axis_index("s")
    b = sid
    h = cid
    r0 = h * HALF
    t0 = h * TE

    bufs = (buf0, buf1)
    in_sems = (in0_sem, in1_sem)
    out_sems = (out0_sem, out1_sem)

    def in_copy(i, par):
        return pltpu.make_async_copy(
            x_hbm.at[b, pl.ds(r0 + i * C, C), :], bufs[par], in_sems[par]
        )

    def out_copy(i, par):
        return pltpu.make_async_copy(
            bufs[par], out_hbm.at[b, pl.ds(r0 + i * C, C), :], out_sems[par]
        )

    ld_lo = pltpu.make_async_copy(x_hbm.at[b, pl.ds(t0, TE), :], lo_v, tail_sem)
    ld_hi = pltpu.make_async_copy(x_hbm.at[b, pl.ds(HI + t0, TE), :], hi_v, tail_sem)
    ld_lo.start()
    ld_hi.start()

    in_copy(0, 0).start()
    in_copy(1, 1).start()

    def outer(i0, carry):
        in_copy(i0, 0).wait()
        out_copy(i0, 0).start()

        in_copy(i0 + 1, 1).wait()
        out_copy(i0 + 1, 1).start()

        @pl.when(i0 + 2 < NCHUNK)
        def _next_even():
            out_copy(i0, 0).wait()
            in_copy(i0 + 2, 0).start()

        @pl.when(i0 + 3 < NCHUNK)
        def _next_odd():
            out_copy(i0 + 1, 1).wait()
            in_copy(i0 + 3, 1).start()

        return carry

    lax.fori_loop(0, NCHUNK // 2, lambda k, c2: outer(k * 2, c2), 0)

    ld_lo.wait()
    ld_hi.wait()

    def _row(r, carry):
        def _col(ci, c2):
            cc = ci * LANES
            lo_v[r, pl.ds(cc, LANES)] = 0.5 * (
                lo_v[r, pl.ds(cc, LANES)] + hi_v[r, pl.ds(cc, LANES)]
            )
            return c2
        return lax.fori_loop(0, CPR, _col, carry)

    lax.fori_loop(0, TE, _row, 0)

    st = pltpu.make_async_copy(lo_v, out_hbm.at[b, pl.ds(N + t0, TE), :], tail_sem)
    st.start()
    st.wait()

    out_copy(NCHUNK - 2, 0).wait()
    out_copy(NCHUNK - 1, 1).wait()


def kernel(inputs):
    return _sc_unpool(inputs)


# speedup vs baseline: 43.3508x; 1.3009x over previous
import jax
import jax.numpy as jnp
from jax.experimental import pallas as pl

B, N, F = 16, 4096, 512
E = 64
HI = 2048
RB = 1040
NBLK = (N + E) // RB
TAIL_COPY = N - (NBLK - 1) * RB


def _body(x_ref, lo_ref, hi_ref, out_ref):
    j = pl.program_id(1)

    @pl.when(j < NBLK - 1)
    def _copy():
        out_ref[...] = x_ref[...]

    @pl.when(j == NBLK - 1)
    def _tail():
        out_ref[0, :TAIL_COPY, :] = x_ref[0, :TAIL_COPY, :]
        out_ref[0, TAIL_COPY:, :] = 0.5 * (lo_ref[0] + hi_ref[0])


def kernel(inputs):
    grid = (B, NBLK)
    return pl.pallas_call(
        _body,
        grid=grid,
        in_specs=[
            pl.BlockSpec((1, RB, F), lambda b, j: (b, j, 0)),
            pl.BlockSpec((1, E, F), lambda b, j: (b, 0, 0)),
            pl.BlockSpec((1, E, F), lambda b, j: (b, HI // E, 0)),
        ],
        out_specs=pl.BlockSpec((1, RB, F), lambda b, j: (b, j, 0)),
        out_shape=jax.ShapeDtypeStruct((B, N + E, F), inputs.dtype),
    )(inputs, inputs, inputs)
